# baseline (speedup 1.0000x reference)
import functools

import jax
import jax.numpy as jnp
from jax import lax
from jax.experimental import pallas as pl
from jax.experimental.pallas import tpu as pltpu
from jax.experimental.pallas import tpu_sc as plsc

N_CORES = 2
N_SUBCORES = 16
N_TILES = N_CORES * N_SUBCORES
EDGE_BLK = 128
ROW_BLK = 512


def _sc_mesh():
    return plsc.VectorSubcoreMesh(core_axis_name="c", subcore_axis_name="s")


def _rsqrt_newton(x):
    i = plsc.bitcast(x, jnp.int32)
    i = jnp.int32(0x5F3759DF) - (i >> 1)
    y = plsc.bitcast(i, jnp.float32)
    for _ in range(3):
        y = y * (1.5 - 0.5 * x * y * y)
    return y


def _make_degree(n_pad, e_pad):
    cpt = e_pad // (N_SUBCORES * EDGE_BLK)
    rpt = n_pad // N_SUBCORES

    @functools.partial(
        pl.kernel,
        mesh=_sc_mesh(),
        out_type=jax.ShapeDtypeStruct((n_pad * 16,), jnp.float32),
        scratch_types=[
            pltpu.VMEM((e_pad // N_SUBCORES,), jnp.int32),
            pltpu.VMEM((n_pad,), jnp.float32),
            pltpu.VMEM((rpt,), jnp.float32),
            pltpu.VMEM((rpt * 16,), jnp.float32),
            pltpu.VMEM_SHARED((N_SUBCORES, n_pad), jnp.float32),
        ],
        compiler_params=pltpu.CompilerParams(needs_layout_passes=False),
    )
    def deg_kernel(dst_hbm, out_hbm, didx, hist, mbuf, dis16, shared):
        c = lax.axis_index("c")
        s = lax.axis_index("s")
        zero16 = jnp.zeros((16,), jnp.float32)
        ones16 = jnp.ones((16,), jnp.float32)

        def zbody(i, carry):
            hist[pl.ds(i * 16, 16)] = zero16
            return carry

        lax.fori_loop(0, n_pad // 16, zbody, 0)
        pltpu.sync_copy(dst_hbm.at[pl.ds(s * cpt * EDGE_BLK, cpt * EDGE_BLK)], didx)

        def body(j, carry):
            base = j * EDGE_BLK
            for k in range(EDGE_BLK // 16):
                idx = didx[pl.ds(base + k * 16, 16)]
                plsc.addupdate_scatter(hist, [idx], ones16)
            return carry

        lax.fori_loop(0, cpt, body, 0)
        pltpu.sync_copy(hist, shared.at[s])
        plsc.subcore_barrier()
        r0 = s * rpt

        def zbody2(i, carry):
            mbuf[pl.ds(i * 16, 16)] = zero16
            return carry

        lax.fori_loop(0, rpt // 16, zbody2, 0)
        for k in range(N_SUBCORES):
            pltpu.sync_copy(shared.at[k, pl.ds(r0, rpt)], hist.at[pl.ds(0, rpt)])

            def abody(i, carry):
                mbuf[pl.ds(i * 16, 16)] = mbuf[pl.ds(i * 16, 16)] + hist[pl.ds(i * 16, 16)]
                return carry

            lax.fori_loop(0, rpt // 16, abody, 0)

        def rbody(i, carry):
            mbuf[pl.ds(i * 16, 16)] = _rsqrt_newton(mbuf[pl.ds(i * 16, 16)] + 1.0)
            return carry

        lax.fori_loop(0, rpt // 16, rbody, 0)

        def bbody(i, carry):
            v = plsc.load_gather(mbuf, [jnp.full((16,), 1, jnp.int32) * i])
            dis16[pl.ds(i * 16, 16)] = v
            return carry

        lax.fori_loop(0, rpt, bbody, 0)

        @pl.when(c == 0)
        def _():
            pltpu.sync_copy(dis16, out_hbm.at[pl.ds(r0 * 16, rpt * 16)])

    return deg_kernel


def _make_propagate(n_pad, d, e_pad):
    cpt = e_pad // (N_TILES * EDGE_BLK)
    rpt = n_pad // N_SUBCORES

    cpt2 = cpt // 2

    @functools.partial(
        pl.kernel,
        mesh=_sc_mesh(),
        out_type=jax.ShapeDtypeStruct((N_CORES, n_pad, d), jnp.float32),
        scratch_types=[
            pltpu.VMEM((cpt2, EDGE_BLK), jnp.int32),
            pltpu.VMEM((cpt2, EDGE_BLK), jnp.int32),
            pltpu.VMEM((2, EDGE_BLK, d), jnp.float32),
            pltpu.VMEM_SHARED((n_pad, d), jnp.float32),
            pltpu.SemaphoreType.DMA((2,)),
            pltpu.SemaphoreType.DMA((2,)),
        ],
    )
    def prop_kernel(g_hbm, src_hbm, dst_hbm, out_hbm, sidx, didx, rows, acc, gsem, ssem):
        c = lax.axis_index("c")
        s = lax.axis_index("s")
        wid = c * N_SUBCORES + s
        r0 = s * rpt
        row_base = wid * cpt
        pltpu.sync_copy(g_hbm.at[pl.ds(r0, rpt)], acc.at[pl.ds(r0, rpt)])
        plsc.subcore_barrier()

        def _run_halves():
         for h in range(2):
            pltpu.sync_copy(src_hbm.at[pl.ds(row_base + h * cpt2, cpt2)], sidx)
            pltpu.sync_copy(dst_hbm.at[pl.ds(row_base + h * cpt2, cpt2)], didx)
            pltpu.async_copy(g_hbm.at[sidx.at[0]], rows.at[0], gsem.at[0])

            def body(g, carry):
                for b in range(2):
                    j = g * 2 + b
                    pltpu.make_async_copy(g_hbm.at[sidx.at[j]], rows.at[b], gsem.at[b]).wait()

                    @pl.when(j + 1 < cpt2)
                    def _():
                        @pl.when(j >= 1)
                        def _():
                            pltpu.make_async_copy(
                                rows.at[1 - b], acc.at[didx.at[j - 1]], ssem.at[1 - b]
                            ).wait()

                        pltpu.async_copy(g_hbm.at[sidx.at[j + 1]], rows.at[1 - b], gsem.at[1 - b])

                    pltpu.async_copy(rows.at[b], acc.at[didx.at[j]], ssem.at[b], add=True)
                return carry

            lax.fori_loop(0, cpt2 // 2, body, 0)
            pltpu.make_async_copy(rows.at[0], acc.at[didx.at[cpt2 - 2]], ssem.at[0]).wait()
            pltpu.make_async_copy(rows.at[1], acc.at[didx.at[cpt2 - 1]], ssem.at[1]).wait()

        pl.when(c == 1)(_run_halves)

        plsc.subcore_barrier()
        pltpu.sync_copy(acc.at[pl.ds(r0, rpt)], out_hbm.at[c, pl.ds(r0, rpt)])

    return prop_kernel


def _prep_body(dis_ref, x_ref, w1_ref, g1_ref):
    h = jnp.dot(x_ref[...], w1_ref[...], preferred_element_type=jnp.float32)
    g1_ref[...] = dis_ref[...][:, 0:1] * h


def _tc_prep(dis, x_p, w1, n_pad):
    df, dh = w1.shape
    return pl.pallas_call(
        _prep_body,
        grid=(n_pad // ROW_BLK,),
        in_specs=[
            pl.BlockSpec((ROW_BLK, 16), lambda i: (i, 0)),
            pl.BlockSpec((ROW_BLK, df), lambda i: (i, 0)),
            pl.BlockSpec((df, dh), lambda i: (0, 0)),
        ],
        out_specs=pl.BlockSpec((ROW_BLK, dh), lambda i: (i, 0)),
        out_shape=jax.ShapeDtypeStruct((n_pad, dh), jnp.float32),
    )(dis, x_p, w1)


def _mid_body(accA_ref, accB_ref, g1_ref, dis_ref, b1_ref, u_ref):
    d = dis_ref[...][:, 0:1]
    t = (accA_ref[...] + accB_ref[...] - g1_ref[...]) * d + b1_ref[0:1, :]
    u_ref[...] = jnp.maximum(t, 0.0) * d


def _tc_mid(accA, accB, g1, dis, b1_2d, n_pad):
    dh = g1.shape[1]
    return pl.pallas_call(
        _mid_body,
        grid=(n_pad // ROW_BLK,),
        in_specs=[
            pl.BlockSpec((ROW_BLK, dh), lambda i: (i, 0)),
            pl.BlockSpec((ROW_BLK, dh), lambda i: (i, 0)),
            pl.BlockSpec((ROW_BLK, dh), lambda i: (i, 0)),
            pl.BlockSpec((ROW_BLK, 16), lambda i: (i, 0)),
            pl.BlockSpec((8, dh), lambda i: (0, 0)),
        ],
        out_specs=pl.BlockSpec((ROW_BLK, dh), lambda i: (i, 0)),
        out_shape=jax.ShapeDtypeStruct((n_pad, dh), jnp.float32),
    )(accA, accB, g1, dis, b1_2d)


def _final_body(accA_ref, accB_ref, u_ref, dis_ref, b2_ref, w2_ref, out_ref):
    d = dis_ref[...][:, 0:1]
    v = accA_ref[...] + accB_ref[...] - u_ref[...]
    out_ref[...] = d * jnp.dot(v, w2_ref[...], preferred_element_type=jnp.float32) + b2_ref[0:1, :]


def _tc_final(accA, accB, u, dis, b2_2d, w2, n_pad):
    dh, dc = w2.shape
    return pl.pallas_call(
        _final_body,
        grid=(n_pad // ROW_BLK,),
        in_specs=[
            pl.BlockSpec((ROW_BLK, dh), lambda i: (i, 0)),
            pl.BlockSpec((ROW_BLK, dh), lambda i: (i, 0)),
            pl.BlockSpec((ROW_BLK, dh), lambda i: (i, 0)),
            pl.BlockSpec((ROW_BLK, 16), lambda i: (i, 0)),
            pl.BlockSpec((8, dc), lambda i: (0, 0)),
            pl.BlockSpec((dh, dc), lambda i: (0, 0)),
        ],
        out_specs=pl.BlockSpec((ROW_BLK, dc), lambda i: (i, 0)),
        out_shape=jax.ShapeDtypeStruct((n_pad, dc), jnp.float32),
    )(accA, accB, u, dis, b2_2d, w2)


def kernel(x, edge_index, W1, b1, W2, b2):
    n, df = x.shape
    dh = W1.shape[1]
    dc = W2.shape[1]
    n_pad = -(-n // ROW_BLK) * ROW_BLK

    src = edge_index[0].astype(jnp.int32)
    dst = edge_index[1].astype(jnp.int32)
    e = src.shape[0]
    step = N_TILES * EDGE_BLK * 4
    e_pad = -(-e // step) * step
    if e_pad != e:
        fill = jnp.full((e_pad - e,), n, jnp.int32)
        src = jnp.concatenate([src, fill])
        dst = jnp.concatenate([dst, fill])
    src2d = src.reshape(e_pad // EDGE_BLK, EDGE_BLK)
    dst2d = dst.reshape(e_pad // EDGE_BLK, EDGE_BLK)

    x_p = jnp.pad(x, ((0, n_pad - n), (0, 0)))

    dis = _make_degree(n_pad, e_pad)(dst).reshape(n_pad, 16)
    g1 = _tc_prep(dis, x_p, W1, n_pad)
    prop = _make_propagate(n_pad, dh, e_pad)
    acc1 = prop(g1, src2d, dst2d)
    u = _tc_mid(acc1[0], acc1[1], g1, dis, jnp.broadcast_to(b1, (8, dh)), n_pad)
    acc2 = prop(u, src2d, dst2d)
    out = _tc_final(acc2[0], acc2[1], u, dis, jnp.broadcast_to(b2, (8, dc)), W2, n_pad)
    return out[:n]


# speedup vs baseline: 1.1130x; 1.1130x over previous
import functools

import jax
import jax.numpy as jnp
from jax import lax
from jax.experimental import pallas as pl
from jax.experimental.pallas import tpu as pltpu
from jax.experimental.pallas import tpu_sc as plsc

N_CORES = 2
N_SUBCORES = 16
N_TILES = N_CORES * N_SUBCORES
EDGE_BLK = 128
ROW_BLK = 512


def _sc_mesh():
    return plsc.VectorSubcoreMesh(core_axis_name="c", subcore_axis_name="s")


def _rsqrt_newton(x):
    i = plsc.bitcast(x, jnp.int32)
    i = jnp.int32(0x5F3759DF) - (i >> 1)
    y = plsc.bitcast(i, jnp.float32)
    for _ in range(3):
        y = y * (1.5 - 0.5 * x * y * y)
    return y


def _make_degree(n_pad, e_pad):
    cpt = e_pad // (N_SUBCORES * EDGE_BLK)
    rpt = n_pad // N_SUBCORES

    @functools.partial(
        pl.kernel,
        mesh=_sc_mesh(),
        out_type=jax.ShapeDtypeStruct((n_pad * 16,), jnp.float32),
        scratch_types=[
            pltpu.VMEM((e_pad // N_SUBCORES,), jnp.int32),
            pltpu.VMEM((n_pad,), jnp.float32),
            pltpu.VMEM((rpt,), jnp.float32),
            pltpu.VMEM((rpt * 16,), jnp.float32),
            pltpu.VMEM_SHARED((N_SUBCORES, n_pad), jnp.float32),
        ],
        compiler_params=pltpu.CompilerParams(needs_layout_passes=False),
    )
    def deg_kernel(dst_hbm, out_hbm, didx, hist, mbuf, dis16, shared):
        c = lax.axis_index("c")
        s = lax.axis_index("s")
        zero16 = jnp.zeros((16,), jnp.float32)
        ones16 = jnp.ones((16,), jnp.float32)

        def zbody(i, carry):
            hist[pl.ds(i * 16, 16)] = zero16
            return carry

        lax.fori_loop(0, n_pad // 16, zbody, 0)
        pltpu.sync_copy(dst_hbm.at[pl.ds(s * cpt * EDGE_BLK, cpt * EDGE_BLK)], didx)

        def body(j, carry):
            base = j * EDGE_BLK
            for k in range(EDGE_BLK // 16):
                idx = didx[pl.ds(base + k * 16, 16)]
                plsc.addupdate_scatter(hist, [idx], ones16)
            return carry

        lax.fori_loop(0, cpt, body, 0)
        pltpu.sync_copy(hist, shared.at[s])
        plsc.subcore_barrier()
        r0 = s * rpt

        def zbody2(i, carry):
            mbuf[pl.ds(i * 16, 16)] = zero16
            return carry

        lax.fori_loop(0, rpt // 16, zbody2, 0)
        for k in range(N_SUBCORES):
            pltpu.sync_copy(shared.at[k, pl.ds(r0, rpt)], hist.at[pl.ds(0, rpt)])

            def abody(i, carry):
                mbuf[pl.ds(i * 16, 16)] = mbuf[pl.ds(i * 16, 16)] + hist[pl.ds(i * 16, 16)]
                return carry

            lax.fori_loop(0, rpt // 16, abody, 0)

        def rbody(i, carry):
            mbuf[pl.ds(i * 16, 16)] = _rsqrt_newton(mbuf[pl.ds(i * 16, 16)] + 1.0)
            return carry

        lax.fori_loop(0, rpt // 16, rbody, 0)

        def bbody(i, carry):
            v = plsc.load_gather(mbuf, [jnp.full((16,), 1, jnp.int32) * i])
            dis16[pl.ds(i * 16, 16)] = v
            return carry

        lax.fori_loop(0, rpt, bbody, 0)

        @pl.when(c == 0)
        def _():
            pltpu.sync_copy(dis16, out_hbm.at[pl.ds(r0 * 16, rpt * 16)])

    return deg_kernel


STAGE = 32
CORE0_STAGES = 4
CPTP = STAGE * (CORE0_STAGES + 1)


def _make_propagate(n_pad, d, e_pad):
    rpt = n_pad // N_SUBCORES
    assert e_pad % (N_SUBCORES * CPTP * EDGE_BLK) == 0
    tiles_chunks = e_pad // (N_SUBCORES * EDGE_BLK)
    rep = tiles_chunks // CPTP

    @functools.partial(
        pl.kernel,
        mesh=_sc_mesh(),
        out_type=jax.ShapeDtypeStruct((N_CORES, n_pad, d), jnp.float32),
        scratch_types=[
            pltpu.VMEM((STAGE, EDGE_BLK), jnp.int32),
            pltpu.VMEM((STAGE, EDGE_BLK), jnp.int32),
            pltpu.VMEM((2, EDGE_BLK, d), jnp.float32),
            pltpu.VMEM_SHARED((n_pad, d), jnp.float32),
            pltpu.SemaphoreType.DMA((2,)),
            pltpu.SemaphoreType.DMA((2,)),
        ],
    )
    def prop_kernel(g_hbm, src_hbm, dst_hbm, out_hbm, sidx, didx, rows, acc, gsem, ssem):
        c = lax.axis_index("c")
        s = lax.axis_index("s")
        r0 = s * rpt
        pltpu.sync_copy(g_hbm.at[pl.ds(r0, rpt)], acc.at[pl.ds(r0, rpt)])
        plsc.subcore_barrier()
        my_base = s * tiles_chunks + jnp.where(c == 0, 0, CORE0_STAGES * STAGE) * rep

        def stage(base):
            pltpu.sync_copy(src_hbm.at[pl.ds(base, STAGE)], sidx)
            pltpu.sync_copy(dst_hbm.at[pl.ds(base, STAGE)], didx)
            pltpu.async_copy(g_hbm.at[sidx.at[0]], rows.at[0], gsem.at[0])

            def body(g, carry):
                for b in range(2):
                    j = g * 2 + b
                    pltpu.make_async_copy(g_hbm.at[sidx.at[j]], rows.at[b], gsem.at[b]).wait()

                    @pl.when(j + 1 < STAGE)
                    def _():
                        @pl.when(j >= 1)
                        def _():
                            pltpu.make_async_copy(
                                rows.at[1 - b], acc.at[didx.at[j - 1]], ssem.at[1 - b]
                            ).wait()

                        pltpu.async_copy(g_hbm.at[sidx.at[j + 1]], rows.at[1 - b], gsem.at[1 - b])

                    pltpu.async_copy(rows.at[b], acc.at[didx.at[j]], ssem.at[b], add=True)
                return carry

            lax.fori_loop(0, STAGE // 2, body, 0)
            pltpu.make_async_copy(rows.at[0], acc.at[didx.at[STAGE - 2]], ssem.at[0]).wait()
            pltpu.make_async_copy(rows.at[1], acc.at[didx.at[STAGE - 1]], ssem.at[1]).wait()

        for st in range(CORE0_STAGES * rep):

            @pl.when((c == 0) | (st < rep))
            def _():
                stage(my_base + st * STAGE)

        plsc.subcore_barrier()
        pltpu.sync_copy(acc.at[pl.ds(r0, rpt)], out_hbm.at[c, pl.ds(r0, rpt)])

    return prop_kernel


def _prep_body(dis_ref, x_ref, w1_ref, g1_ref):
    h = jnp.dot(x_ref[...], w1_ref[...], preferred_element_type=jnp.float32)
    g1_ref[...] = dis_ref[...][:, 0:1] * h


def _tc_prep(dis, x_p, w1, n_pad):
    df, dh = w1.shape
    return pl.pallas_call(
        _prep_body,
        grid=(n_pad // ROW_BLK,),
        in_specs=[
            pl.BlockSpec((ROW_BLK, 16), lambda i: (i, 0)),
            pl.BlockSpec((ROW_BLK, df), lambda i: (i, 0)),
            pl.BlockSpec((df, dh), lambda i: (0, 0)),
        ],
        out_specs=pl.BlockSpec((ROW_BLK, dh), lambda i: (i, 0)),
        out_shape=jax.ShapeDtypeStruct((n_pad, dh), jnp.float32),
    )(dis, x_p, w1)


def _mid_body(accA_ref, accB_ref, g1_ref, dis_ref, b1_ref, u_ref):
    d = dis_ref[...][:, 0:1]
    t = (accA_ref[...] + accB_ref[...] - g1_ref[...]) * d + b1_ref[0:1, :]
    u_ref[...] = jnp.maximum(t, 0.0) * d


def _tc_mid(accA, accB, g1, dis, b1_2d, n_pad):
    dh = g1.shape[1]
    return pl.pallas_call(
        _mid_body,
        grid=(n_pad // ROW_BLK,),
        in_specs=[
            pl.BlockSpec((ROW_BLK, dh), lambda i: (i, 0)),
            pl.BlockSpec((ROW_BLK, dh), lambda i: (i, 0)),
            pl.BlockSpec((ROW_BLK, dh), lambda i: (i, 0)),
            pl.BlockSpec((ROW_BLK, 16), lambda i: (i, 0)),
            pl.BlockSpec((8, dh), lambda i: (0, 0)),
        ],
        out_specs=pl.BlockSpec((ROW_BLK, dh), lambda i: (i, 0)),
        out_shape=jax.ShapeDtypeStruct((n_pad, dh), jnp.float32),
    )(accA, accB, g1, dis, b1_2d)


def _final_body(accA_ref, accB_ref, u_ref, dis_ref, b2_ref, w2_ref, out_ref):
    d = dis_ref[...][:, 0:1]
    v = accA_ref[...] + accB_ref[...] - u_ref[...]
    out_ref[...] = d * jnp.dot(v, w2_ref[...], preferred_element_type=jnp.float32) + b2_ref[0:1, :]


def _tc_final(accA, accB, u, dis, b2_2d, w2, n_pad):
    dh, dc = w2.shape
    return pl.pallas_call(
        _final_body,
        grid=(n_pad // ROW_BLK,),
        in_specs=[
            pl.BlockSpec((ROW_BLK, dh), lambda i: (i, 0)),
            pl.BlockSpec((ROW_BLK, dh), lambda i: (i, 0)),
            pl.BlockSpec((ROW_BLK, dh), lambda i: (i, 0)),
            pl.BlockSpec((ROW_BLK, 16), lambda i: (i, 0)),
            pl.BlockSpec((8, dc), lambda i: (0, 0)),
            pl.BlockSpec((dh, dc), lambda i: (0, 0)),
        ],
        out_specs=pl.BlockSpec((ROW_BLK, dc), lambda i: (i, 0)),
        out_shape=jax.ShapeDtypeStruct((n_pad, dc), jnp.float32),
    )(accA, accB, u, dis, b2_2d, w2)


def kernel(x, edge_index, W1, b1, W2, b2):
    n, df = x.shape
    dh = W1.shape[1]
    dc = W2.shape[1]
    n_pad = -(-n // ROW_BLK) * ROW_BLK

    src = edge_index[0].astype(jnp.int32)
    dst = edge_index[1].astype(jnp.int32)
    e = src.shape[0]
    step = N_SUBCORES * CPTP * EDGE_BLK
    e_pad = -(-e // step) * step
    if e_pad != e:
        fill = jnp.full((e_pad - e,), n, jnp.int32)
        src = jnp.concatenate([src, fill])
        dst = jnp.concatenate([dst, fill])
    src2d = src.reshape(e_pad // EDGE_BLK, EDGE_BLK)
    dst2d = dst.reshape(e_pad // EDGE_BLK, EDGE_BLK)

    x_p = jnp.pad(x, ((0, n_pad - n), (0, 0)))

    dis = _make_degree(n_pad, e_pad)(dst).reshape(n_pad, 16)
    g1 = _tc_prep(dis, x_p, W1, n_pad)
    prop = _make_propagate(n_pad, dh, e_pad)
    acc1 = prop(g1, src2d, dst2d)
    u = _tc_mid(acc1[0], acc1[1], g1, dis, jnp.broadcast_to(b1, (8, dh)), n_pad)
    acc2 = prop(u, src2d, dst2d)
    out = _tc_final(acc2[0], acc2[1], u, dis, jnp.broadcast_to(b2, (8, dc)), W2, n_pad)
    return out[:n]


# speedup vs baseline: 2.7118x; 2.4365x over previous
import functools

import jax
import jax.numpy as jnp
from jax import lax
from jax.experimental import pallas as pl
from jax.experimental.pallas import tpu as pltpu
from jax.experimental.pallas import tpu_sc as plsc

N_CORES = 2
N_SUBCORES = 16
N_TILES = N_CORES * N_SUBCORES
EDGE_BLK = 128
ROW_BLK = 512


def _sc_mesh():
    return plsc.VectorSubcoreMesh(core_axis_name="c", subcore_axis_name="s")


def _rsqrt_newton(x):
    i = plsc.bitcast(x, jnp.int32)
    i = jnp.int32(0x5F3759DF) - (i >> 1)
    y = plsc.bitcast(i, jnp.float32)
    for _ in range(3):
        y = y * (1.5 - 0.5 * x * y * y)
    return y


def _make_degree(n_pad, e_pad):
    cpt = e_pad // (N_SUBCORES * EDGE_BLK)
    rpt = n_pad // N_SUBCORES

    @functools.partial(
        pl.kernel,
        mesh=_sc_mesh(),
        out_type=jax.ShapeDtypeStruct((n_pad * 16,), jnp.float32),
        scratch_types=[
            pltpu.VMEM((e_pad // N_SUBCORES,), jnp.int32),
            pltpu.VMEM((n_pad,), jnp.float32),
            pltpu.VMEM((rpt,), jnp.float32),
            pltpu.VMEM((rpt * 16,), jnp.float32),
            pltpu.VMEM_SHARED((N_SUBCORES, n_pad), jnp.float32),
        ],
        compiler_params=pltpu.CompilerParams(needs_layout_passes=False),
    )
    def deg_kernel(dst_hbm, out_hbm, didx, hist, mbuf, dis16, shared):
        c = lax.axis_index("c")
        s = lax.axis_index("s")
        zero16 = jnp.zeros((16,), jnp.float32)
        ones16 = jnp.ones((16,), jnp.float32)

        def zbody(i, carry):
            hist[pl.ds(i * 16, 16)] = zero16
            return carry

        lax.fori_loop(0, n_pad // 16, zbody, 0)
        pltpu.sync_copy(dst_hbm.at[pl.ds(s * cpt * EDGE_BLK, cpt * EDGE_BLK)], didx)

        def body(j, carry):
            base = j * EDGE_BLK
            for k in range(EDGE_BLK // 16):
                idx = didx[pl.ds(base + k * 16, 16)]
                plsc.addupdate_scatter(hist, [idx], ones16)
            return carry

        lax.fori_loop(0, cpt, body, 0)
        pltpu.sync_copy(hist, shared.at[s])
        plsc.subcore_barrier()
        r0 = s * rpt

        def zbody2(i, carry):
            mbuf[pl.ds(i * 16, 16)] = zero16
            return carry

        lax.fori_loop(0, rpt // 16, zbody2, 0)
        for k in range(N_SUBCORES):
            pltpu.sync_copy(shared.at[k, pl.ds(r0, rpt)], hist.at[pl.ds(0, rpt)])

            def abody(i, carry):
                mbuf[pl.ds(i * 16, 16)] = mbuf[pl.ds(i * 16, 16)] + hist[pl.ds(i * 16, 16)]
                return carry

            lax.fori_loop(0, rpt // 16, abody, 0)

        def rbody(i, carry):
            mbuf[pl.ds(i * 16, 16)] = _rsqrt_newton(mbuf[pl.ds(i * 16, 16)] + 1.0)
            return carry

        lax.fori_loop(0, rpt // 16, rbody, 0)

        def bbody(i, carry):
            v = plsc.load_gather(mbuf, [jnp.full((16,), 1, jnp.int32) * i])
            dis16[pl.ds(i * 16, 16)] = v
            return carry

        lax.fori_loop(0, rpt, bbody, 0)

        @pl.when(c == 0)
        def _():
            pltpu.sync_copy(dis16, out_hbm.at[pl.ds(r0 * 16, rpt * 16)])

    return deg_kernel


STAGE = 40
S0 = 2
S1 = 2
CPTP = STAGE * (S0 + S1)


def _make_propagate(n_pad, d, e_pad):
    rpt = n_pad // N_SUBCORES
    assert e_pad % (N_SUBCORES * CPTP * EDGE_BLK) == 0
    tiles_chunks = e_pad // (N_SUBCORES * EDGE_BLK)
    rep = tiles_chunks // CPTP

    @functools.partial(
        pl.kernel,
        mesh=_sc_mesh(),
        out_type=jax.ShapeDtypeStruct((N_CORES, n_pad, d), jnp.float32),
        scratch_types=[
            pltpu.VMEM((STAGE, EDGE_BLK), jnp.int32),
            pltpu.VMEM((STAGE, EDGE_BLK), jnp.int32),
            pltpu.VMEM((2, EDGE_BLK, d), jnp.float32),
            pltpu.VMEM_SHARED((n_pad, d), jnp.float32),
            pltpu.SemaphoreType.DMA((2,)),
            pltpu.SemaphoreType.DMA((2,)),
        ],
    )
    def prop_kernel(g_hbm, src_hbm, dst_hbm, out_hbm, sidx, didx, rows, acc, gsem, ssem):
        c = lax.axis_index("c")
        s = lax.axis_index("s")
        r0 = s * rpt
        pltpu.sync_copy(g_hbm.at[pl.ds(r0, rpt)], acc.at[pl.ds(r0, rpt)])
        plsc.subcore_barrier()
        my_base = s * tiles_chunks + jnp.where(c == 0, 0, S0 * STAGE) * rep

        def stage(base):
            pltpu.sync_copy(src_hbm.at[pl.ds(base, STAGE)], sidx)
            pltpu.sync_copy(dst_hbm.at[pl.ds(base, STAGE)], didx)
            pltpu.async_copy(g_hbm.at[sidx.at[0]], rows.at[0], gsem.at[0])

            def body(g, carry):
                for b in range(2):
                    j = g * 2 + b
                    pltpu.make_async_copy(g_hbm.at[sidx.at[j]], rows.at[b], gsem.at[b]).wait()

                    @pl.when(j + 1 < STAGE)
                    def _():
                        @pl.when(j >= 1)
                        def _():
                            pltpu.make_async_copy(
                                rows.at[1 - b], acc.at[didx.at[j - 1]], ssem.at[1 - b]
                            ).wait()

                        pltpu.async_copy(g_hbm.at[sidx.at[j + 1]], rows.at[1 - b], gsem.at[1 - b])

                    pltpu.async_copy(rows.at[b], acc.at[didx.at[j]], ssem.at[b], add=True)
                return carry

            lax.fori_loop(0, STAGE // 2, body, 0)
            pltpu.make_async_copy(rows.at[0], acc.at[didx.at[STAGE - 2]], ssem.at[0]).wait()
            pltpu.make_async_copy(rows.at[1], acc.at[didx.at[STAGE - 1]], ssem.at[1]).wait()

        for st in range(S0 * rep):

            @pl.when((c == 0) | (st < S1 * rep))
            def _():
                stage(my_base + st * STAGE)

        plsc.subcore_barrier()
        pltpu.sync_copy(acc.at[pl.ds(r0, rpt)], out_hbm.at[c, pl.ds(r0, rpt)])

    return prop_kernel


def _prep_body(dis_ref, x_ref, w1_ref, g1_ref):
    h = jnp.dot(x_ref[...], w1_ref[...], preferred_element_type=jnp.float32)
    g1_ref[...] = dis_ref[...][:, 0:1] * h


def _tc_prep(dis, x_p, w1, n_pad):
    df, dh = w1.shape
    return pl.pallas_call(
        _prep_body,
        grid=(n_pad // ROW_BLK,),
        in_specs=[
            pl.BlockSpec((ROW_BLK, 16), lambda i: (i, 0)),
            pl.BlockSpec((ROW_BLK, df), lambda i: (i, 0)),
            pl.BlockSpec((df, dh), lambda i: (0, 0)),
        ],
        out_specs=pl.BlockSpec((ROW_BLK, dh), lambda i: (i, 0)),
        out_shape=jax.ShapeDtypeStruct((n_pad, dh), jnp.float32),
    )(dis, x_p, w1)


def _mid_body(accA_ref, accB_ref, g1_ref, dis_ref, b1_ref, u_ref):
    d = dis_ref[...][:, 0:1]
    t = (accA_ref[...] + accB_ref[...] - g1_ref[...]) * d + b1_ref[0:1, :]
    u_ref[...] = jnp.maximum(t, 0.0) * d


def _tc_mid(accA, accB, g1, dis, b1_2d, n_pad):
    dh = g1.shape[1]
    return pl.pallas_call(
        _mid_body,
        grid=(n_pad // ROW_BLK,),
        in_specs=[
            pl.BlockSpec((ROW_BLK, dh), lambda i: (i, 0)),
            pl.BlockSpec((ROW_BLK, dh), lambda i: (i, 0)),
            pl.BlockSpec((ROW_BLK, dh), lambda i: (i, 0)),
            pl.BlockSpec((ROW_BLK, 16), lambda i: (i, 0)),
            pl.BlockSpec((8, dh), lambda i: (0, 0)),
        ],
        out_specs=pl.BlockSpec((ROW_BLK, dh), lambda i: (i, 0)),
        out_shape=jax.ShapeDtypeStruct((n_pad, dh), jnp.float32),
    )(accA, accB, g1, dis, b1_2d)


def _final_body(accA_ref, accB_ref, u_ref, dis_ref, b2_ref, w2_ref, out_ref):
    d = dis_ref[...][:, 0:1]
    v = accA_ref[...] + accB_ref[...] - u_ref[...]
    out_ref[...] = d * jnp.dot(v, w2_ref[...], preferred_element_type=jnp.float32) + b2_ref[0:1, :]


def _tc_final(accA, accB, u, dis, b2_2d, w2, n_pad):
    dh, dc = w2.shape
    return pl.pallas_call(
        _final_body,
        grid=(n_pad // ROW_BLK,),
        in_specs=[
            pl.BlockSpec((ROW_BLK, dh), lambda i: (i, 0)),
            pl.BlockSpec((ROW_BLK, dh), lambda i: (i, 0)),
            pl.BlockSpec((ROW_BLK, dh), lambda i: (i, 0)),
            pl.BlockSpec((ROW_BLK, 16), lambda i: (i, 0)),
            pl.BlockSpec((8, dc), lambda i: (0, 0)),
            pl.BlockSpec((dh, dc), lambda i: (0, 0)),
        ],
        out_specs=pl.BlockSpec((ROW_BLK, dc), lambda i: (i, 0)),
        out_shape=jax.ShapeDtypeStruct((n_pad, dc), jnp.float32),
    )(accA, accB, u, dis, b2_2d, w2)


def kernel(x, edge_index, W1, b1, W2, b2):
    n, df = x.shape
    dh = W1.shape[1]
    dc = W2.shape[1]
    n_pad = -(-(n + 1) // ROW_BLK) * ROW_BLK

    src = edge_index[0].astype(jnp.int32)
    dst = edge_index[1].astype(jnp.int32)
    e = src.shape[0]
    step = N_SUBCORES * CPTP * EDGE_BLK
    e_pad = -(-e // step) * step
    if e_pad != e:
        fill = n + jnp.arange(e_pad - e, dtype=jnp.int32) % (n_pad - n)
        src = jnp.concatenate([src, fill])
        dst = jnp.concatenate([dst, fill])
    src2d = src.reshape(e_pad // EDGE_BLK, EDGE_BLK)
    dst2d = dst.reshape(e_pad // EDGE_BLK, EDGE_BLK)

    x_p = jnp.pad(x, ((0, n_pad - n), (0, 0)))

    dis = _make_degree(n_pad, e_pad)(dst).reshape(n_pad, 16)
    g1 = _tc_prep(dis, x_p, W1, n_pad)
    prop = _make_propagate(n_pad, dh, e_pad)
    acc1 = prop(g1, src2d, dst2d)
    u = _tc_mid(acc1[0], acc1[1], g1, dis, jnp.broadcast_to(b1, (8, dh)), n_pad)
    acc2 = prop(u, src2d, dst2d)
    out = _tc_final(acc2[0], acc2[1], u, dis, jnp.broadcast_to(b2, (8, dc)), W2, n_pad)
    return out[:n]
